# baseline (speedup 1.0000x reference)
import functools

import jax
import jax.numpy as jnp
from jax import lax
from jax.experimental import pallas as pl
from jax.experimental.pallas import tpu as pltpu
from jax.experimental.pallas import tpu_sc as plsc

B, T, DIM = 1, 2048, 768
BT = B * T
FD = 2 * DIM
BANK, RANK, TOP_K = 2048, 8, 4
ROW = RANK * FD
PROW = ROW // 2
PR = DIM
THRESH = 0.99

TB_A = 256


def _scores_topk_body(x_ref, kt_ref, idx_ref, w_ref):
    s = jnp.dot(x_ref[...], kt_ref[...], preferred_element_type=jnp.float32)
    lane = lax.broadcasted_iota(jnp.int32, (TB_A, BANK), 1)
    vals, idxs = [], []
    for k in range(TOP_K):
        m = jnp.max(s, axis=1)
        ik = jnp.min(jnp.where(s == m[:, None], lane, BANK), axis=1)
        vals.append(m)
        idxs.append(ik)
        s = jnp.where(lane == ik[:, None], -jnp.inf, s)
    es = [jnp.exp(v - vals[0]) for v in vals]
    tot = es[0] + es[1] + es[2] + es[3]
    lane8 = lax.broadcasted_iota(jnp.int32, (TB_A, 8), 1)
    lane64 = lax.broadcasted_iota(jnp.int32, (TB_A, TOP_K * 16), 1) // 16
    idx_out = jnp.zeros((TB_A, 8), jnp.int32)
    w_out = jnp.zeros((TB_A, TOP_K * 16), jnp.int32)
    for k in range(TOP_K):
        wq = jnp.round(es[k] / tot * (2.0 ** W_BITS)).astype(jnp.int32)
        idx_out = jnp.where(lane8 == k, idxs[k][:, None], idx_out)
        w_out = jnp.where(lane64 == k, wq[:, None], w_out)
    idx_ref[...] = idx_out
    w_ref[...] = w_out


def _scores_topk(x, kt, nt=BT):
    grid = nt // TB_A
    return pl.pallas_call(
        _scores_topk_body,
        grid=(grid,),
        in_specs=[
            pl.BlockSpec((TB_A, FD), lambda i: (i, 0)),
            pl.BlockSpec((FD, BANK), lambda i: (0, 0)),
        ],
        out_specs=[
            pl.BlockSpec((TB_A, 8), lambda i: (i, 0)),
            pl.BlockSpec((TB_A, TOP_K * 16), lambda i: (i, 0)),
        ],
        out_shape=[
            jax.ShapeDtypeStruct((nt, 8), jnp.int32),
            jax.ShapeDtypeStruct((nt, TOP_K * 16), jnp.int32),
        ],
        compiler_params=pltpu.CompilerParams(
            dimension_semantics=("arbitrary",)),
    )(x, kt)


SC_CORES, SC_SUBCORES = 2, 16
NW = SC_CORES * SC_SUBCORES
TPW = BT // NW

V_BITS = 16
W_BITS = 14
_WH = 1 << (W_BITS - 1)


def _gather(v, idx, w, nt=BT):
    tpw = nt // NW

    def _gather_body(v_hbm, idx_hbm, w_hbm, out_hbm,
                     idx_v, w_v, rows_v, acc_v,
                     sem_i0, sem_i1, sem_o0, sem_o1):
        wid = lax.axis_index("s") * SC_CORES + lax.axis_index("c")
        base = wid * tpw
        sem_i = (sem_i0, sem_i1)
        sem_o = (sem_o0, sem_o1)

        pltpu.sync_copy(idx_hbm.at[pl.ds(base * 8, tpw * 8)], idx_v)
        pltpu.sync_copy(
            w_hbm.at[pl.ds(base * TOP_K * 16, tpw * TOP_K * 16)], w_v)

        def cp_in(t, buf):
            return pltpu.make_async_copy(
                v_hbm.at[idx_v.at[pl.ds(t * 8, TOP_K)]], rows_v.at[buf],
                sem_i[buf])

        def cp_out(t, buf):
            return pltpu.make_async_copy(
                acc_v.at[buf], out_hbm.at[base + t], sem_o[buf])

        cp_in(0, 0).start()

        @pl.loop(0, tpw, step=2)
        def _token_pair(t):
            for b in (0, 1):
                tok = t + b

                @pl.when(tok + 1 < tpw)
                def _():
                    cp_in(tok + 1, 1 - b).start()

                cp_in(tok, b).wait()

                @pl.when(tok >= 2)
                def _():
                    cp_out(tok - 2, b).wait()

                wv = [w_v[pl.ds((tok * TOP_K + k) * 16, 16)]
                      for k in range(TOP_K)]

                @pl.loop(0, PROW // 16, unroll=8)
                def _chunk(j):
                    sl = pl.ds(j * 16, 16)
                    sa = jnp.zeros((16,), jnp.int32)
                    sb = jnp.zeros((16,), jnp.int32)
                    for k in range(TOP_K):
                        r = rows_v[b, k, sl]
                        sa = sa + ((r << 16) >> 16) * wv[k]
                        sb = sb + (r >> 16) * wv[k]
                    sa = (sa + _WH) >> W_BITS
                    sb = (sb + _WH) >> W_BITS
                    acc_v[b, sl] = (sb << 16) | (sa & 0xFFFF)

                cp_out(tok, b).start()

        cp_out(tpw - 2, 0).wait()
        cp_out(tpw - 1, 1).wait()

    return pl.kernel(
        _gather_body,
        out_type=jax.ShapeDtypeStruct((nt, PROW), jnp.int32),
        mesh=plsc.VectorSubcoreMesh(core_axis_name="c", subcore_axis_name="s",
                                    num_cores=SC_CORES,
                                    num_subcores=SC_SUBCORES),
        scratch_types=[
            pltpu.VMEM((tpw * 8,), jnp.int32),
            pltpu.VMEM((tpw * TOP_K * 16,), jnp.int32),
            pltpu.VMEM((2, TOP_K, PROW), jnp.int32),
            pltpu.VMEM((2, PROW), jnp.int32),
            pltpu.SemaphoreType.DMA,
            pltpu.SemaphoreType.DMA,
            pltpu.SemaphoreType.DMA,
            pltpu.SemaphoreType.DMA,
        ],
    )(v, idx, w)


TB_C = 128


def _proj_body(final, u_ref, p_ref, hre_ref, him_ref, hbl_ref, *rest):
    if final:
        ph0_ref, x0_ref, scal_ref, out_ref, ph_ref = rest
    else:
        out_ref, ph_ref = rest
    pr = p_ref[:, :DIM]
    pi = p_ref[:, DIM:]
    ar = jnp.zeros((TB_C, DIM), jnp.float32)
    ai = jnp.zeros((TB_C, DIM), jnp.float32)
    inv_s2 = 2.0 ** (-2 * V_BITS)
    for r in range(RANK):
        rp = u_ref[:, r * PR:(r + 1) * PR]
        ur = ((rp << 16) >> 16).astype(jnp.float32)
        ui = (rp >> 16).astype(jnp.float32)
        nrm2 = jnp.sum(ur * ur + ui * ui, axis=1) * inv_s2
        q = inv_s2 / jnp.maximum(nrm2, 1e-6)
        cr = jnp.sum(ur * pr + ui * pi, axis=1) * q
        ci = jnp.sum(ur * pi - ui * pr, axis=1) * q
        ar = ar + cr[:, None] * ur - ci[:, None] * ui
        ai = ai + ci[:, None] * ur + cr[:, None] * ui
    sq = jnp.sum(ar * ar + ai * ai, axis=1)
    scale = lax.rsqrt(jnp.maximum(sq, 1e-6))
    psr = ar * scale[:, None]
    psi_ = ai * scale[:, None]
    ls = [jnp.sum(psr * hre_ref[j][None, :] + psi_ * him_ref[j][None, :],
                  axis=1) + hbl_ref[j]
          for j in range(3)]
    m = jnp.maximum(jnp.maximum(ls[0], ls[1]), ls[2])
    e0 = jnp.exp(ls[0] - m)
    ph = e0 / (e0 + jnp.exp(ls[1] - m) + jnp.exp(ls[2] - m))
    ph_ref[...] = ph
    psin = jnp.concatenate([psr, psi_], axis=1)
    if final:
        ph0 = ph0_ref[...]
        still = (ph0 < THRESH).astype(jnp.float32)
        w_a = jnp.where(ph0 >= THRESH, 1.0, ph0)
        w_b = (1.0 - ph0) * still
        acc = (w_a[:, None] * p_ref[...] + w_b[:, None] * psin) * scal_ref[1]
        x0 = x0_ref[...]
        out_ref[...] = x0 + scal_ref[0] * (acc - x0)
    else:
        out_ref[...] = psin


def _project(u, p, hre, him, hbl, ph0=None, x0=None, scal=None, nt=BT):
    final = ph0 is not None
    grid = nt // TB_C
    tok2 = pl.BlockSpec((TB_C, FD), lambda i: (i, 0))
    in_specs = [
        pl.BlockSpec((TB_C, PROW), lambda i: (i, 0)),
        tok2,
        pl.BlockSpec((8, DIM), lambda i: (0, 0)),
        pl.BlockSpec((8, DIM), lambda i: (0, 0)),
        pl.BlockSpec(memory_space=pltpu.SMEM),
    ]
    args = [u, p, hre, him, hbl]
    if final:
        in_specs += [pl.BlockSpec((TB_C,), lambda i: (i,)), tok2,
                     pl.BlockSpec(memory_space=pltpu.SMEM)]
        args += [ph0, x0, scal]
    return pl.pallas_call(
        functools.partial(_proj_body, final),
        grid=(grid,),
        in_specs=in_specs,
        out_specs=[tok2, pl.BlockSpec((TB_C,), lambda i: (i,))],
        out_shape=[
            jax.ShapeDtypeStruct((nt, FD), jnp.float32),
            jax.ShapeDtypeStruct((nt,), jnp.float32),
        ],
        compiler_params=pltpu.CompilerParams(
            dimension_semantics=("arbitrary",)),
    )(*args)


def _route(idx8, w8, nt):
    return idx8.reshape(nt * 8), w8.reshape(nt * TOP_K * 16)


def kernel(psi, bank_keys, bank_values, halt_w_logits, halt_b_logits,
           halt_w_abg, halt_b_abg, head_mix, out_scale):
    psi3 = psi.reshape(BT, DIM, 2)
    x0 = jnp.concatenate([psi3[..., 0], psi3[..., 1]], axis=1)
    k2 = jnp.concatenate([bank_keys[..., 0], bank_keys[..., 1]], axis=1)
    kt = k2.T
    vi = jnp.clip(jnp.round(bank_values.reshape(BANK, PROW, 2)
                            * (2.0 ** V_BITS)), -32767, 32767).astype(jnp.int32)
    v = (vi[..., 1] << 16) | (vi[..., 0] & 0xFFFF)
    hwl3 = halt_w_logits.reshape(DIM, 2, 3)
    hre = jnp.zeros((8, DIM), jnp.float32).at[:3].set(hwl3[:, 0, :].T)
    him = jnp.zeros((8, DIM), jnp.float32).at[:3].set(hwl3[:, 1, :].T)
    head_w = jax.nn.softmax(head_mix)[0]
    scal = jnp.stack([out_scale.astype(jnp.float32), head_w])

    NCH = 4
    HN = BT // NCH
    hbl = halt_b_logits

    idx8, w8 = _scores_topk(x0, kt)
    idxf, wef = _route(idx8, w8, BT)

    outs, ph0s, ph1s = [], [], []
    halves = []
    for h in range(NCH):
        u1 = _gather(v, idxf[h * HN * 8:(h + 1) * HN * 8],
                     wef[h * HN * 64:(h + 1) * HN * 64], nt=HN)
        x0_h = x0[h * HN:(h + 1) * HN]
        psi1_h, ph0_h = _project(u1, x0_h, hre, him, hbl, nt=HN)
        halves.append((x0_h, psi1_h, ph0_h))
    for h in range(NCH):
        x0_h, psi1_h, ph0_h = halves[h]
        idx8b, w8b = _scores_topk(psi1_h, kt, nt=HN)
        idxb, web = _route(idx8b, w8b, HN)
        u2 = _gather(v, idxb, web, nt=HN)
        out_h, ph1_h = _project(u2, psi1_h, hre, him, hbl,
                                ph0=ph0_h, x0=x0_h, scal=scal, nt=HN)
        outs.append(out_h)
        ph0s.append(ph0_h)
        ph1s.append(ph1_h)

    ph0 = jnp.concatenate(ph0s)
    ph1 = jnp.concatenate(ph1s)
    psi_out = jnp.concatenate(outs, axis=0)
    still = (ph0 < THRESH).astype(jnp.float32)
    cost = jnp.mean(ph0 + ph1 * still)
    out3 = jnp.stack([psi_out[:, :DIM], psi_out[:, DIM:]], axis=-1)
    return out3.reshape(B, T, DIM, 2), cost


# speedup vs baseline: 1.0251x; 1.0251x over previous
import functools

import jax
import jax.numpy as jnp
from jax import lax
from jax.experimental import pallas as pl
from jax.experimental.pallas import tpu as pltpu
from jax.experimental.pallas import tpu_sc as plsc

B, T, DIM = 1, 2048, 768
BT = B * T
FD = 2 * DIM
BANK, RANK, TOP_K = 2048, 8, 4
ROW = RANK * FD
PROW = ROW // 2
PR = DIM
THRESH = 0.99

TB_A = 256


def _scores_topk_body(x_ref, kt_ref, idx_ref, w_ref):
    s = jnp.dot(x_ref[...], kt_ref[...], preferred_element_type=jnp.float32)
    lane = lax.broadcasted_iota(jnp.int32, (TB_A, BANK), 1)
    vals, idxs = [], []
    for k in range(TOP_K):
        m = jnp.max(s, axis=1)
        ik = jnp.min(jnp.where(s == m[:, None], lane, BANK), axis=1)
        vals.append(m)
        idxs.append(ik)
        s = jnp.where(lane == ik[:, None], -jnp.inf, s)
    es = [jnp.exp(v - vals[0]) for v in vals]
    tot = es[0] + es[1] + es[2] + es[3]
    lane8 = lax.broadcasted_iota(jnp.int32, (TB_A, 8), 1)
    lane64 = lax.broadcasted_iota(jnp.int32, (TB_A, TOP_K * 16), 1) // 16
    idx_out = jnp.zeros((TB_A, 8), jnp.int32)
    w_out = jnp.zeros((TB_A, TOP_K * 16), jnp.int32)
    for k in range(TOP_K):
        wq = jnp.round(es[k] / tot * (2.0 ** W_BITS)).astype(jnp.int32)
        idx_out = jnp.where(lane8 == k, idxs[k][:, None], idx_out)
        w_out = jnp.where(lane64 == k, wq[:, None], w_out)
    idx_ref[...] = idx_out
    w_ref[...] = w_out


def _scores_topk(x, kt, nt=BT):
    grid = nt // TB_A
    return pl.pallas_call(
        _scores_topk_body,
        grid=(grid,),
        in_specs=[
            pl.BlockSpec((TB_A, FD), lambda i: (i, 0)),
            pl.BlockSpec((FD, BANK), lambda i: (0, 0)),
        ],
        out_specs=[
            pl.BlockSpec((TB_A, 8), lambda i: (i, 0)),
            pl.BlockSpec((TB_A, TOP_K * 16), lambda i: (i, 0)),
        ],
        out_shape=[
            jax.ShapeDtypeStruct((nt, 8), jnp.int32),
            jax.ShapeDtypeStruct((nt, TOP_K * 16), jnp.int32),
        ],
        compiler_params=pltpu.CompilerParams(
            dimension_semantics=("arbitrary",)),
    )(x, kt)


SC_CORES, SC_SUBCORES = 2, 16
NW = SC_CORES * SC_SUBCORES
TPW = BT // NW

V_BITS = 16
W_BITS = 14
_WH = 1 << (W_BITS - 1)


def _gather(v, idx, w, nt=BT):
    tpw = nt // NW

    def _gather_body(v_hbm, idx_hbm, w_hbm, out_hbm,
                     idx_v, w_v, rows_v, acc_v,
                     sem_i0, sem_i1, sem_o0, sem_o1):
        wid = lax.axis_index("s") * SC_CORES + lax.axis_index("c")
        base = wid * tpw
        sem_i = (sem_i0, sem_i1)
        sem_o = (sem_o0, sem_o1)

        pltpu.sync_copy(idx_hbm.at[pl.ds(base * 8, tpw * 8)], idx_v)
        pltpu.sync_copy(
            w_hbm.at[pl.ds(base * TOP_K * 16, tpw * TOP_K * 16)], w_v)

        def cp_in(t, buf):
            return pltpu.make_async_copy(
                v_hbm.at[idx_v.at[pl.ds(t * 8, TOP_K)]], rows_v.at[buf],
                sem_i[buf])

        def cp_out(t, buf):
            return pltpu.make_async_copy(
                acc_v.at[buf], out_hbm.at[base + t], sem_o[buf])

        cp_in(0, 0).start()

        @pl.loop(0, tpw, step=2)
        def _token_pair(t):
            for b in (0, 1):
                tok = t + b

                @pl.when(tok + 1 < tpw)
                def _():
                    cp_in(tok + 1, 1 - b).start()

                cp_in(tok, b).wait()

                @pl.when(tok >= 2)
                def _():
                    cp_out(tok - 2, b).wait()

                wv = [w_v[pl.ds((tok * TOP_K + k) * 16, 16)]
                      for k in range(TOP_K)]

                @pl.loop(0, PROW // 16, unroll=8)
                def _chunk(j):
                    sl = pl.ds(j * 16, 16)
                    sa = jnp.zeros((16,), jnp.int32)
                    sb = jnp.zeros((16,), jnp.int32)
                    for k in range(TOP_K):
                        r = rows_v[b, k, sl]
                        sa = sa + ((r << 16) >> 16) * wv[k]
                        sb = sb + (r >> 16) * wv[k]
                    sa = (sa + _WH) >> W_BITS
                    sb = (sb + _WH) >> W_BITS
                    acc_v[b, sl] = (sb << 16) | (sa & 0xFFFF)

                cp_out(tok, b).start()

        cp_out(tpw - 2, 0).wait()
        cp_out(tpw - 1, 1).wait()

    return pl.kernel(
        _gather_body,
        out_type=jax.ShapeDtypeStruct((nt, PROW), jnp.int32),
        mesh=plsc.VectorSubcoreMesh(core_axis_name="c", subcore_axis_name="s",
                                    num_cores=SC_CORES,
                                    num_subcores=SC_SUBCORES),
        scratch_types=[
            pltpu.VMEM((tpw * 8,), jnp.int32),
            pltpu.VMEM((tpw * TOP_K * 16,), jnp.int32),
            pltpu.VMEM((2, TOP_K, PROW), jnp.int32),
            pltpu.VMEM((2, PROW), jnp.int32),
            pltpu.SemaphoreType.DMA,
            pltpu.SemaphoreType.DMA,
            pltpu.SemaphoreType.DMA,
            pltpu.SemaphoreType.DMA,
        ],
    )(v, idx, w)


TB_C = 128


def _proj_body(final, u_ref, p_ref, hre_ref, him_ref, hbl_ref, *rest):
    if final:
        ph0_ref, x0_ref, scal_ref, out_ref, ph_ref = rest
    else:
        out_ref, ph_ref = rest
    pr = p_ref[:, :DIM]
    pi = p_ref[:, DIM:]
    ar = jnp.zeros((TB_C, DIM), jnp.float32)
    ai = jnp.zeros((TB_C, DIM), jnp.float32)
    inv_s2 = 2.0 ** (-2 * V_BITS)
    for r in range(RANK):
        rp = u_ref[:, r * PR:(r + 1) * PR]
        ur = ((rp << 16) >> 16).astype(jnp.float32)
        ui = (rp >> 16).astype(jnp.float32)
        nrm2 = jnp.sum(ur * ur + ui * ui, axis=1) * inv_s2
        q = inv_s2 / jnp.maximum(nrm2, 1e-6)
        cr = jnp.sum(ur * pr + ui * pi, axis=1) * q
        ci = jnp.sum(ur * pi - ui * pr, axis=1) * q
        ar = ar + cr[:, None] * ur - ci[:, None] * ui
        ai = ai + ci[:, None] * ur + cr[:, None] * ui
    sq = jnp.sum(ar * ar + ai * ai, axis=1)
    scale = lax.rsqrt(jnp.maximum(sq, 1e-6))
    psr = ar * scale[:, None]
    psi_ = ai * scale[:, None]
    ls = [jnp.sum(psr * hre_ref[j][None, :] + psi_ * him_ref[j][None, :],
                  axis=1) + hbl_ref[j]
          for j in range(3)]
    m = jnp.maximum(jnp.maximum(ls[0], ls[1]), ls[2])
    e0 = jnp.exp(ls[0] - m)
    ph = e0 / (e0 + jnp.exp(ls[1] - m) + jnp.exp(ls[2] - m))
    ph_ref[...] = ph
    psin = jnp.concatenate([psr, psi_], axis=1)
    if final:
        ph0 = ph0_ref[...]
        still = (ph0 < THRESH).astype(jnp.float32)
        w_a = jnp.where(ph0 >= THRESH, 1.0, ph0)
        w_b = (1.0 - ph0) * still
        acc = (w_a[:, None] * p_ref[...] + w_b[:, None] * psin) * scal_ref[1]
        x0 = x0_ref[...]
        out_ref[...] = x0 + scal_ref[0] * (acc - x0)
    else:
        out_ref[...] = psin


def _project(u, p, hre, him, hbl, ph0=None, x0=None, scal=None, nt=BT):
    final = ph0 is not None
    grid = nt // TB_C
    tok2 = pl.BlockSpec((TB_C, FD), lambda i: (i, 0))
    in_specs = [
        pl.BlockSpec((TB_C, PROW), lambda i: (i, 0)),
        tok2,
        pl.BlockSpec((8, DIM), lambda i: (0, 0)),
        pl.BlockSpec((8, DIM), lambda i: (0, 0)),
        pl.BlockSpec(memory_space=pltpu.SMEM),
    ]
    args = [u, p, hre, him, hbl]
    if final:
        in_specs += [pl.BlockSpec((TB_C,), lambda i: (i,)), tok2,
                     pl.BlockSpec(memory_space=pltpu.SMEM)]
        args += [ph0, x0, scal]
    return pl.pallas_call(
        functools.partial(_proj_body, final),
        grid=(grid,),
        in_specs=in_specs,
        out_specs=[tok2, pl.BlockSpec((TB_C,), lambda i: (i,))],
        out_shape=[
            jax.ShapeDtypeStruct((nt, FD), jnp.float32),
            jax.ShapeDtypeStruct((nt,), jnp.float32),
        ],
        compiler_params=pltpu.CompilerParams(
            dimension_semantics=("arbitrary",)),
    )(*args)


def _route(idx8, w8, nt):
    return idx8.reshape(nt * 8), w8.reshape(nt * TOP_K * 16)


def kernel(psi, bank_keys, bank_values, halt_w_logits, halt_b_logits,
           halt_w_abg, halt_b_abg, head_mix, out_scale):
    psi3 = psi.reshape(BT, DIM, 2)
    x0 = jnp.concatenate([psi3[..., 0], psi3[..., 1]], axis=1)
    k2 = jnp.concatenate([bank_keys[..., 0], bank_keys[..., 1]], axis=1)
    kt = k2.T
    vi = jnp.clip(jnp.round(bank_values.reshape(BANK, PROW, 2)
                            * (2.0 ** V_BITS)), -32767, 32767).astype(jnp.int32)
    v = (vi[..., 1] << 16) | (vi[..., 0] & 0xFFFF)
    hwl3 = halt_w_logits.reshape(DIM, 2, 3)
    hre = jnp.zeros((8, DIM), jnp.float32).at[:3].set(hwl3[:, 0, :].T)
    him = jnp.zeros((8, DIM), jnp.float32).at[:3].set(hwl3[:, 1, :].T)
    head_w = jax.nn.softmax(head_mix)[0]
    scal = jnp.stack([out_scale.astype(jnp.float32), head_w])

    NCH = 2
    HN = BT // NCH
    hbl = halt_b_logits

    idx8, w8 = _scores_topk(x0, kt)
    idxf, wef = _route(idx8, w8, BT)

    outs, ph0s, ph1s = [], [], []
    halves = []
    for h in range(NCH):
        u1 = _gather(v, idxf[h * HN * 8:(h + 1) * HN * 8],
                     wef[h * HN * 64:(h + 1) * HN * 64], nt=HN)
        x0_h = x0[h * HN:(h + 1) * HN]
        psi1_h, ph0_h = _project(u1, x0_h, hre, him, hbl, nt=HN)
        halves.append((x0_h, psi1_h, ph0_h))
    for h in range(NCH):
        x0_h, psi1_h, ph0_h = halves[h]
        idx8b, w8b = _scores_topk(psi1_h, kt, nt=HN)
        idxb, web = _route(idx8b, w8b, HN)
        u2 = _gather(v, idxb, web, nt=HN)
        out_h, ph1_h = _project(u2, psi1_h, hre, him, hbl,
                                ph0=ph0_h, x0=x0_h, scal=scal, nt=HN)
        outs.append(out_h)
        ph0s.append(ph0_h)
        ph1s.append(ph1_h)

    ph0 = jnp.concatenate(ph0s)
    ph1 = jnp.concatenate(ph1s)
    psi_out = jnp.concatenate(outs, axis=0)
    still = (ph0 < THRESH).astype(jnp.float32)
    cost = jnp.mean(ph0 + ph1 * still)
    out3 = jnp.stack([psi_out[:, :DIM], psi_out[:, DIM:]], axis=-1)
    return out3.reshape(B, T, DIM, 2), cost


# speedup vs baseline: 1.0488x; 1.0230x over previous
import functools

import jax
import jax.numpy as jnp
from jax import lax
from jax.experimental import pallas as pl
from jax.experimental.pallas import tpu as pltpu
from jax.experimental.pallas import tpu_sc as plsc

B, T, DIM = 1, 2048, 768
BT = B * T
FD = 2 * DIM
BANK, RANK, TOP_K = 2048, 8, 4
ROW = RANK * FD
PROW = ROW // 2
PR = DIM
THRESH = 0.99

TB_A = 256


def _scores_topk_body(x_ref, kt_ref, idx_ref, w_ref):
    s = jnp.dot(x_ref[...], kt_ref[...], preferred_element_type=jnp.float32)
    lane = lax.broadcasted_iota(jnp.int32, (TB_A, BANK), 1)
    vals, idxs = [], []
    for k in range(TOP_K):
        m = jnp.max(s, axis=1)
        ik = jnp.min(jnp.where(s == m[:, None], lane, BANK), axis=1)
        vals.append(m)
        idxs.append(ik)
        s = jnp.where(lane == ik[:, None], -jnp.inf, s)
    es = [jnp.exp(v - vals[0]) for v in vals]
    tot = es[0] + es[1] + es[2] + es[3]
    lane8 = lax.broadcasted_iota(jnp.int32, (TB_A, 8), 1)
    lane64 = lax.broadcasted_iota(jnp.int32, (TB_A, TOP_K * 16), 1) // 16
    idx_out = jnp.zeros((TB_A, 8), jnp.int32)
    w_out = jnp.zeros((TB_A, TOP_K * 16), jnp.int32)
    for k in range(TOP_K):
        wq = jnp.round(es[k] / tot * (2.0 ** W_BITS)).astype(jnp.int32)
        idx_out = jnp.where(lane8 == k, idxs[k][:, None], idx_out)
        w_out = jnp.where(lane64 == k, wq[:, None], w_out)
    idx_ref[...] = idx_out
    w_ref[...] = w_out


def _scores_topk(x, kt, nt=BT):
    grid = nt // TB_A
    return pl.pallas_call(
        _scores_topk_body,
        grid=(grid,),
        in_specs=[
            pl.BlockSpec((TB_A, FD), lambda i: (i, 0)),
            pl.BlockSpec((FD, BANK), lambda i: (0, 0)),
        ],
        out_specs=[
            pl.BlockSpec((TB_A, 8), lambda i: (i, 0)),
            pl.BlockSpec((TB_A, TOP_K * 16), lambda i: (i, 0)),
        ],
        out_shape=[
            jax.ShapeDtypeStruct((nt, 8), jnp.int32),
            jax.ShapeDtypeStruct((nt, TOP_K * 16), jnp.int32),
        ],
        compiler_params=pltpu.CompilerParams(
            dimension_semantics=("arbitrary",)),
    )(x, kt)


SC_CORES, SC_SUBCORES = 2, 16
NW = SC_CORES * SC_SUBCORES
TPW = BT // NW

V_BITS = 16
W_BITS = 14
_WH = 1 << (W_BITS - 1)


def _gather(v, idx, w, nt=BT):
    tpw = nt // NW

    def _gather_body(v_hbm, idx_hbm, w_hbm, out_hbm,
                     idx_v, w_v, rows_v, acc_v,
                     sem_i0, sem_i1, sem_o0, sem_o1):
        wid = lax.axis_index("s") * SC_CORES + lax.axis_index("c")
        base = wid * tpw
        sem_i = (sem_i0, sem_i1)
        sem_o = (sem_o0, sem_o1)

        pltpu.sync_copy(idx_hbm.at[pl.ds(base * 8, tpw * 8)], idx_v)
        pltpu.sync_copy(
            w_hbm.at[pl.ds(base * TOP_K * 16, tpw * TOP_K * 16)], w_v)

        def cp_in(t, buf):
            return pltpu.make_async_copy(
                v_hbm.at[idx_v.at[pl.ds(t * 8, TOP_K)]], rows_v.at[buf],
                sem_i[buf])

        def cp_out(t, buf):
            return pltpu.make_async_copy(
                acc_v.at[buf], out_hbm.at[base + t], sem_o[buf])

        cp_in(0, 0).start()

        @pl.loop(0, tpw, step=2)
        def _token_pair(t):
            for b in (0, 1):
                tok = t + b

                @pl.when(tok + 1 < tpw)
                def _():
                    cp_in(tok + 1, 1 - b).start()

                cp_in(tok, b).wait()

                @pl.when(tok >= 2)
                def _():
                    cp_out(tok - 2, b).wait()

                wv = [w_v[pl.ds((tok * TOP_K + k) * 16, 16)]
                      for k in range(TOP_K)]

                @pl.loop(0, PROW // 16, unroll=8)
                def _chunk(j):
                    sl = pl.ds(j * 16, 16)
                    sa = jnp.zeros((16,), jnp.int32)
                    sb = jnp.zeros((16,), jnp.int32)
                    for k in range(TOP_K):
                        r = rows_v[b, k, sl]
                        sa = sa + ((r << 16) >> 16) * wv[k]
                        sb = sb + (r >> 16) * wv[k]
                    sa = sa >> W_BITS
                    sb = sb >> W_BITS
                    acc_v[b, sl] = (sb << 16) | (sa & 0xFFFF)

                cp_out(tok, b).start()

        cp_out(tpw - 2, 0).wait()
        cp_out(tpw - 1, 1).wait()

    return pl.kernel(
        _gather_body,
        out_type=jax.ShapeDtypeStruct((nt, PROW), jnp.int32),
        mesh=plsc.VectorSubcoreMesh(core_axis_name="c", subcore_axis_name="s",
                                    num_cores=SC_CORES,
                                    num_subcores=SC_SUBCORES),
        scratch_types=[
            pltpu.VMEM((tpw * 8,), jnp.int32),
            pltpu.VMEM((tpw * TOP_K * 16,), jnp.int32),
            pltpu.VMEM((2, TOP_K, PROW), jnp.int32),
            pltpu.VMEM((2, PROW), jnp.int32),
            pltpu.SemaphoreType.DMA,
            pltpu.SemaphoreType.DMA,
            pltpu.SemaphoreType.DMA,
            pltpu.SemaphoreType.DMA,
        ],
    )(v, idx, w)


TB_C = 256


def _proj_body(final, u_ref, p_ref, hre_ref, him_ref, hbl_ref, *rest):
    if final:
        ph0_ref, x0_ref, scal_ref, out_ref, ph_ref = rest
    else:
        out_ref, ph_ref = rest
    pr = p_ref[:, :DIM]
    pi = p_ref[:, DIM:]
    ar = jnp.zeros((TB_C, DIM), jnp.float32)
    ai = jnp.zeros((TB_C, DIM), jnp.float32)
    inv_s2 = 2.0 ** (-2 * V_BITS)
    for r in range(RANK):
        rp = u_ref[:, r * PR:(r + 1) * PR]
        ur = ((rp << 16) >> 16).astype(jnp.float32)
        ui = (rp >> 16).astype(jnp.float32)
        nrm2 = jnp.sum(ur * ur + ui * ui, axis=1) * inv_s2
        q = inv_s2 / jnp.maximum(nrm2, 1e-6)
        cr = jnp.sum(ur * pr + ui * pi, axis=1) * q
        ci = jnp.sum(ur * pi - ui * pr, axis=1) * q
        ar = ar + cr[:, None] * ur - ci[:, None] * ui
        ai = ai + ci[:, None] * ur + cr[:, None] * ui
    sq = jnp.sum(ar * ar + ai * ai, axis=1)
    scale = lax.rsqrt(jnp.maximum(sq, 1e-6))
    psr = ar * scale[:, None]
    psi_ = ai * scale[:, None]
    ls = [jnp.sum(psr * hre_ref[j][None, :] + psi_ * him_ref[j][None, :],
                  axis=1) + hbl_ref[j]
          for j in range(3)]
    m = jnp.maximum(jnp.maximum(ls[0], ls[1]), ls[2])
    e0 = jnp.exp(ls[0] - m)
    ph = e0 / (e0 + jnp.exp(ls[1] - m) + jnp.exp(ls[2] - m))
    ph_ref[...] = ph
    psin = jnp.concatenate([psr, psi_], axis=1)
    if final:
        ph0 = ph0_ref[...]
        still = (ph0 < THRESH).astype(jnp.float32)
        w_a = jnp.where(ph0 >= THRESH, 1.0, ph0)
        w_b = (1.0 - ph0) * still
        acc = (w_a[:, None] * p_ref[...] + w_b[:, None] * psin) * scal_ref[1]
        x0 = x0_ref[...]
        out_ref[...] = x0 + scal_ref[0] * (acc - x0)
    else:
        out_ref[...] = psin


def _project(u, p, hre, him, hbl, ph0=None, x0=None, scal=None, nt=BT):
    final = ph0 is not None
    grid = nt // TB_C
    tok2 = pl.BlockSpec((TB_C, FD), lambda i: (i, 0))
    in_specs = [
        pl.BlockSpec((TB_C, PROW), lambda i: (i, 0)),
        tok2,
        pl.BlockSpec((8, DIM), lambda i: (0, 0)),
        pl.BlockSpec((8, DIM), lambda i: (0, 0)),
        pl.BlockSpec(memory_space=pltpu.SMEM),
    ]
    args = [u, p, hre, him, hbl]
    if final:
        in_specs += [pl.BlockSpec((TB_C,), lambda i: (i,)), tok2,
                     pl.BlockSpec(memory_space=pltpu.SMEM)]
        args += [ph0, x0, scal]
    return pl.pallas_call(
        functools.partial(_proj_body, final),
        grid=(grid,),
        in_specs=in_specs,
        out_specs=[tok2, pl.BlockSpec((TB_C,), lambda i: (i,))],
        out_shape=[
            jax.ShapeDtypeStruct((nt, FD), jnp.float32),
            jax.ShapeDtypeStruct((nt,), jnp.float32),
        ],
        compiler_params=pltpu.CompilerParams(
            dimension_semantics=("arbitrary",)),
    )(*args)


def _route(idx8, w8, nt):
    return idx8.reshape(nt * 8), w8.reshape(nt * TOP_K * 16)


def kernel(psi, bank_keys, bank_values, halt_w_logits, halt_b_logits,
           halt_w_abg, halt_b_abg, head_mix, out_scale):
    psi3 = psi.reshape(BT, DIM, 2)
    x0 = jnp.concatenate([psi3[..., 0], psi3[..., 1]], axis=1)
    k2 = jnp.concatenate([bank_keys[..., 0], bank_keys[..., 1]], axis=1)
    kt = k2.T
    vi = jnp.clip(jnp.round(bank_values.reshape(BANK, PROW, 2)
                            * (2.0 ** V_BITS)), -32767, 32767).astype(jnp.int32)
    v = (vi[..., 1] << 16) | (vi[..., 0] & 0xFFFF)
    hwl3 = halt_w_logits.reshape(DIM, 2, 3)
    hre = jnp.zeros((8, DIM), jnp.float32).at[:3].set(hwl3[:, 0, :].T)
    him = jnp.zeros((8, DIM), jnp.float32).at[:3].set(hwl3[:, 1, :].T)
    head_w = jax.nn.softmax(head_mix)[0]
    scal = jnp.stack([out_scale.astype(jnp.float32), head_w])

    NCH = 2
    HN = BT // NCH
    hbl = halt_b_logits

    idx8, w8 = _scores_topk(x0, kt)
    idxf, wef = _route(idx8, w8, BT)

    outs, ph0s, ph1s = [], [], []
    halves = []
    for h in range(NCH):
        u1 = _gather(v, idxf[h * HN * 8:(h + 1) * HN * 8],
                     wef[h * HN * 64:(h + 1) * HN * 64], nt=HN)
        x0_h = x0[h * HN:(h + 1) * HN]
        psi1_h, ph0_h = _project(u1, x0_h, hre, him, hbl, nt=HN)
        halves.append((x0_h, psi1_h, ph0_h))
    for h in range(NCH):
        x0_h, psi1_h, ph0_h = halves[h]
        idx8b, w8b = _scores_topk(psi1_h, kt, nt=HN)
        idxb, web = _route(idx8b, w8b, HN)
        u2 = _gather(v, idxb, web, nt=HN)
        out_h, ph1_h = _project(u2, psi1_h, hre, him, hbl,
                                ph0=ph0_h, x0=x0_h, scal=scal, nt=HN)
        outs.append(out_h)
        ph0s.append(ph0_h)
        ph1s.append(ph1_h)

    ph0 = jnp.concatenate(ph0s)
    ph1 = jnp.concatenate(ph1s)
    psi_out = jnp.concatenate(outs, axis=0)
    still = (ph0 < THRESH).astype(jnp.float32)
    cost = jnp.mean(ph0 + ph1 * still)
    out3 = jnp.stack([psi_out[:, :DIM], psi_out[:, DIM:]], axis=-1)
    return out3.reshape(B, T, DIM, 2), cost


# speedup vs baseline: 1.0652x; 1.0156x over previous
import functools

import jax
import jax.numpy as jnp
from jax import lax
from jax.experimental import pallas as pl
from jax.experimental.pallas import tpu as pltpu
from jax.experimental.pallas import tpu_sc as plsc

B, T, DIM = 1, 2048, 768
BT = B * T
FD = 2 * DIM
BANK, RANK, TOP_K = 2048, 8, 4
ROW = RANK * FD
PROW = ROW // 2
PR = DIM
THRESH = 0.99

TB_A = 256


def _scores_topk_body(x_ref, kt_ref, idx_ref, w_ref):
    s = jnp.dot(x_ref[...], kt_ref[...], preferred_element_type=jnp.float32)
    lane = lax.broadcasted_iota(jnp.int32, (TB_A, BANK), 1)
    vals, idxs = [], []
    for k in range(TOP_K):
        m = jnp.max(s, axis=1)
        ik = jnp.min(jnp.where(s == m[:, None], lane, BANK), axis=1)
        vals.append(m)
        idxs.append(ik)
        s = jnp.where(lane == ik[:, None], -jnp.inf, s)
    es = [jnp.exp(v - vals[0]) for v in vals]
    tot = es[0] + es[1] + es[2] + es[3]
    lane8 = lax.broadcasted_iota(jnp.int32, (TB_A, 8), 1)
    lane64 = lax.broadcasted_iota(jnp.int32, (TB_A, TOP_K * 16), 1) // 16
    idx_out = jnp.zeros((TB_A, 8), jnp.int32)
    w_out = jnp.zeros((TB_A, TOP_K * 16), jnp.int32)
    for k in range(TOP_K):
        wq = jnp.round(es[k] / tot * (2.0 ** W_BITS)).astype(jnp.int32)
        idx_out = jnp.where(lane8 == k, idxs[k][:, None], idx_out)
        w_out = jnp.where(lane64 == k, wq[:, None], w_out)
    idx_ref[...] = idx_out
    w_ref[...] = w_out


def _scores_topk(x, kt, nt=BT):
    grid = nt // TB_A
    return pl.pallas_call(
        _scores_topk_body,
        grid=(grid,),
        in_specs=[
            pl.BlockSpec((TB_A, FD), lambda i: (i, 0)),
            pl.BlockSpec((FD, BANK), lambda i: (0, 0)),
        ],
        out_specs=[
            pl.BlockSpec((TB_A, 8), lambda i: (i, 0)),
            pl.BlockSpec((TB_A, TOP_K * 16), lambda i: (i, 0)),
        ],
        out_shape=[
            jax.ShapeDtypeStruct((nt, 8), jnp.int32),
            jax.ShapeDtypeStruct((nt, TOP_K * 16), jnp.int32),
        ],
        compiler_params=pltpu.CompilerParams(
            dimension_semantics=("arbitrary",)),
    )(x, kt)


SC_CORES, SC_SUBCORES = 2, 16
NW = SC_CORES * SC_SUBCORES
TPW = BT // NW

V_BITS = 16
W_BITS = 14
_WH = 1 << (W_BITS - 1)


def _gather(v, idx, w, nt=BT):
    tpw = nt // NW

    def _gather_body(v_hbm, idx_hbm, w_hbm, out_hbm,
                     idx_v, w_v, rows_v, acc_v,
                     sem_i0, sem_i1, sem_o0, sem_o1):
        wid = lax.axis_index("s") * SC_CORES + lax.axis_index("c")
        base = wid * tpw
        sem_i = (sem_i0, sem_i1)
        sem_o = (sem_o0, sem_o1)

        pltpu.sync_copy(idx_hbm.at[pl.ds(base * 8, tpw * 8)], idx_v)
        pltpu.sync_copy(
            w_hbm.at[pl.ds(base * TOP_K * 16, tpw * TOP_K * 16)], w_v)

        def cp_in(t, buf):
            return pltpu.make_async_copy(
                v_hbm.at[idx_v.at[pl.ds(t * 8, TOP_K)]], rows_v.at[buf],
                sem_i[buf])

        def cp_out(t, buf):
            return pltpu.make_async_copy(
                acc_v.at[buf], out_hbm.at[base + t], sem_o[buf])

        cp_in(0, 0).start()

        @pl.loop(0, tpw, step=2)
        def _token_pair(t):
            for b in (0, 1):
                tok = t + b

                @pl.when(tok + 1 < tpw)
                def _():
                    cp_in(tok + 1, 1 - b).start()

                cp_in(tok, b).wait()

                @pl.when(tok >= 2)
                def _():
                    cp_out(tok - 2, b).wait()

                wv = [w_v[pl.ds((tok * TOP_K + k) * 16, 16)]
                      for k in range(TOP_K)]

                @pl.loop(0, PROW // 16, unroll=16)
                def _chunk(j):
                    sl = pl.ds(j * 16, 16)
                    sa = jnp.zeros((16,), jnp.int32)
                    sb = jnp.zeros((16,), jnp.int32)
                    for k in range(TOP_K):
                        r = rows_v[b, k, sl]
                        sa = sa + ((r << 16) >> 16) * wv[k]
                        sb = sb + (r >> 16) * wv[k]
                    sa = sa >> W_BITS
                    sb = sb >> W_BITS
                    acc_v[b, sl] = (sb << 16) | (sa & 0xFFFF)

                cp_out(tok, b).start()

        cp_out(tpw - 2, 0).wait()
        cp_out(tpw - 1, 1).wait()

    return pl.kernel(
        _gather_body,
        out_type=jax.ShapeDtypeStruct((nt, PROW), jnp.int32),
        mesh=plsc.VectorSubcoreMesh(core_axis_name="c", subcore_axis_name="s",
                                    num_cores=SC_CORES,
                                    num_subcores=SC_SUBCORES),
        scratch_types=[
            pltpu.VMEM((tpw * 8,), jnp.int32),
            pltpu.VMEM((tpw * TOP_K * 16,), jnp.int32),
            pltpu.VMEM((2, TOP_K, PROW), jnp.int32),
            pltpu.VMEM((2, PROW), jnp.int32),
            pltpu.SemaphoreType.DMA,
            pltpu.SemaphoreType.DMA,
            pltpu.SemaphoreType.DMA,
            pltpu.SemaphoreType.DMA,
        ],
    )(v, idx, w)


TB_C = 256


def _proj_body(final, u_ref, p_ref, hre_ref, him_ref, hbl_ref, *rest):
    if final:
        ph0_ref, x0_ref, scal_ref, out_ref, ph_ref = rest
    else:
        out_ref, ph_ref = rest
    pr = p_ref[:, :DIM]
    pi = p_ref[:, DIM:]
    ar = jnp.zeros((TB_C, DIM), jnp.float32)
    ai = jnp.zeros((TB_C, DIM), jnp.float32)
    inv_s2 = 2.0 ** (-2 * V_BITS)
    for r in range(RANK):
        rp = u_ref[:, r * PR:(r + 1) * PR]
        ur = ((rp << 16) >> 16).astype(jnp.float32)
        ui = (rp >> 16).astype(jnp.float32)
        nrm2 = jnp.sum(ur * ur + ui * ui, axis=1) * inv_s2
        q = inv_s2 / jnp.maximum(nrm2, 1e-6)
        cr = jnp.sum(ur * pr + ui * pi, axis=1) * q
        ci = jnp.sum(ur * pi - ui * pr, axis=1) * q
        ar = ar + cr[:, None] * ur - ci[:, None] * ui
        ai = ai + ci[:, None] * ur + cr[:, None] * ui
    sq = jnp.sum(ar * ar + ai * ai, axis=1)
    scale = lax.rsqrt(jnp.maximum(sq, 1e-6))
    psr = ar * scale[:, None]
    psi_ = ai * scale[:, None]
    ls = [jnp.sum(psr * hre_ref[j][None, :] + psi_ * him_ref[j][None, :],
                  axis=1) + hbl_ref[j]
          for j in range(3)]
    m = jnp.maximum(jnp.maximum(ls[0], ls[1]), ls[2])
    e0 = jnp.exp(ls[0] - m)
    ph = e0 / (e0 + jnp.exp(ls[1] - m) + jnp.exp(ls[2] - m))
    ph_ref[...] = ph
    psin = jnp.concatenate([psr, psi_], axis=1)
    if final:
        ph0 = ph0_ref[...]
        still = (ph0 < THRESH).astype(jnp.float32)
        w_a = jnp.where(ph0 >= THRESH, 1.0, ph0)
        w_b = (1.0 - ph0) * still
        acc = (w_a[:, None] * p_ref[...] + w_b[:, None] * psin) * scal_ref[1]
        x0 = x0_ref[...]
        out_ref[...] = x0 + scal_ref[0] * (acc - x0)
    else:
        out_ref[...] = psin


def _project(u, p, hre, him, hbl, ph0=None, x0=None, scal=None, nt=BT):
    final = ph0 is not None
    grid = nt // TB_C
    tok2 = pl.BlockSpec((TB_C, FD), lambda i: (i, 0))
    in_specs = [
        pl.BlockSpec((TB_C, PROW), lambda i: (i, 0)),
        tok2,
        pl.BlockSpec((8, DIM), lambda i: (0, 0)),
        pl.BlockSpec((8, DIM), lambda i: (0, 0)),
        pl.BlockSpec(memory_space=pltpu.SMEM),
    ]
    args = [u, p, hre, him, hbl]
    if final:
        in_specs += [pl.BlockSpec((TB_C,), lambda i: (i,)), tok2,
                     pl.BlockSpec(memory_space=pltpu.SMEM)]
        args += [ph0, x0, scal]
    return pl.pallas_call(
        functools.partial(_proj_body, final),
        grid=(grid,),
        in_specs=in_specs,
        out_specs=[tok2, pl.BlockSpec((TB_C,), lambda i: (i,))],
        out_shape=[
            jax.ShapeDtypeStruct((nt, FD), jnp.float32),
            jax.ShapeDtypeStruct((nt,), jnp.float32),
        ],
        compiler_params=pltpu.CompilerParams(
            dimension_semantics=("arbitrary",)),
    )(*args)


def _route(idx8, w8, nt):
    return idx8.reshape(nt * 8), w8.reshape(nt * TOP_K * 16)


def kernel(psi, bank_keys, bank_values, halt_w_logits, halt_b_logits,
           halt_w_abg, halt_b_abg, head_mix, out_scale):
    psi3 = psi.reshape(BT, DIM, 2)
    x0 = jnp.concatenate([psi3[..., 0], psi3[..., 1]], axis=1)
    k2 = jnp.concatenate([bank_keys[..., 0], bank_keys[..., 1]], axis=1)
    kt = k2.T
    vi = jnp.clip(jnp.round(bank_values.reshape(BANK, PROW, 2)
                            * (2.0 ** V_BITS)), -32767, 32767).astype(jnp.int32)
    v = (vi[..., 1] << 16) | (vi[..., 0] & 0xFFFF)
    hwl3 = halt_w_logits.reshape(DIM, 2, 3)
    hre = jnp.zeros((8, DIM), jnp.float32).at[:3].set(hwl3[:, 0, :].T)
    him = jnp.zeros((8, DIM), jnp.float32).at[:3].set(hwl3[:, 1, :].T)
    head_w = jax.nn.softmax(head_mix)[0]
    scal = jnp.stack([out_scale.astype(jnp.float32), head_w])

    NCH = 2
    HN = BT // NCH
    hbl = halt_b_logits

    outs, ph0s, ph1s = [], [], []
    halves = []
    for h in range(NCH):
        x0_h = x0[h * HN:(h + 1) * HN]
        idx8, w8 = _scores_topk(x0_h, kt, nt=HN)
        idxf, wef = _route(idx8, w8, HN)
        u1 = _gather(v, idxf, wef, nt=HN)
        psi1_h, ph0_h = _project(u1, x0_h, hre, him, hbl, nt=HN)
        halves.append((x0_h, psi1_h, ph0_h))
    for h in range(NCH):
        x0_h, psi1_h, ph0_h = halves[h]
        idx8b, w8b = _scores_topk(psi1_h, kt, nt=HN)
        idxb, web = _route(idx8b, w8b, HN)
        u2 = _gather(v, idxb, web, nt=HN)
        out_h, ph1_h = _project(u2, psi1_h, hre, him, hbl,
                                ph0=ph0_h, x0=x0_h, scal=scal, nt=HN)
        outs.append(out_h)
        ph0s.append(ph0_h)
        ph1s.append(ph1_h)

    ph0 = jnp.concatenate(ph0s)
    ph1 = jnp.concatenate(ph1s)
    psi_out = jnp.concatenate(outs, axis=0)
    still = (ph0 < THRESH).astype(jnp.float32)
    cost = jnp.mean(ph0 + ph1 * still)
    out3 = jnp.stack([psi_out[:, :DIM], psi_out[:, DIM:]], axis=-1)
    return out3.reshape(B, T, DIM, 2), cost


# speedup vs baseline: 1.0936x; 1.0267x over previous
import functools

import jax
import jax.numpy as jnp
from jax import lax
from jax.experimental import pallas as pl
from jax.experimental.pallas import tpu as pltpu
from jax.experimental.pallas import tpu_sc as plsc

B, T, DIM = 1, 2048, 768
BT = B * T
FD = 2 * DIM
BANK, RANK, TOP_K = 2048, 8, 4
ROW = RANK * FD
PROW = ROW // 2
PR = DIM
THRESH = 0.99

TB_A = 256


def _scores_topk_body(x_ref, kt_ref, idx_ref, w_ref):
    s = jnp.dot(x_ref[...], kt_ref[...], preferred_element_type=jnp.float32)
    lane = lax.broadcasted_iota(jnp.int32, (TB_A, BANK), 1)
    vals, idxs = [], []
    for k in range(TOP_K):
        m = jnp.max(s, axis=1)
        ik = jnp.min(jnp.where(s == m[:, None], lane, BANK), axis=1)
        vals.append(m)
        idxs.append(ik)
        s = jnp.where(lane == ik[:, None], -jnp.inf, s)
    es = [jnp.exp(v - vals[0]) for v in vals]
    tot = es[0] + es[1] + es[2] + es[3]
    lane8 = lax.broadcasted_iota(jnp.int32, (TB_A, 8), 1)
    lane64 = lax.broadcasted_iota(jnp.int32, (TB_A, TOP_K * 16), 1) // 16
    idx_out = jnp.zeros((TB_A, 8), jnp.int32)
    w_out = jnp.zeros((TB_A, TOP_K * 16), jnp.int32)
    wqs = [jnp.round(es[k] / tot * (2.0 ** W_BITS)).astype(jnp.int32)
           for k in range(1, TOP_K)]
    wqs = [(1 << W_BITS) - wqs[0] - wqs[1] - wqs[2]] + wqs
    for k in range(TOP_K):
        idx_out = jnp.where(lane8 == k, idxs[k][:, None], idx_out)
        w_out = jnp.where(lane64 == k, wqs[k][:, None], w_out)
    idx_ref[...] = idx_out
    w_ref[...] = w_out


def _scores_topk(x, kt, nt=BT):
    grid = nt // TB_A
    return pl.pallas_call(
        _scores_topk_body,
        grid=(grid,),
        in_specs=[
            pl.BlockSpec((TB_A, FD), lambda i: (i, 0)),
            pl.BlockSpec((FD, BANK), lambda i: (0, 0)),
        ],
        out_specs=[
            pl.BlockSpec((TB_A, 8), lambda i: (i, 0)),
            pl.BlockSpec((TB_A, TOP_K * 16), lambda i: (i, 0)),
        ],
        out_shape=[
            jax.ShapeDtypeStruct((nt, 8), jnp.int32),
            jax.ShapeDtypeStruct((nt, TOP_K * 16), jnp.int32),
        ],
        compiler_params=pltpu.CompilerParams(
            dimension_semantics=("arbitrary",)),
    )(x, kt)


SC_CORES, SC_SUBCORES = 2, 16
NW = SC_CORES * SC_SUBCORES
TPW = BT // NW

V_BITS = 16
W_BITS = 14
_WH = 1 << (W_BITS - 1)


def _gather(v, idx, w, nt=BT):
    tpw = nt // NW

    def _gather_body(v_hbm, idx_hbm, w_hbm, out_hbm,
                     idx_v, w_v, rows_v, acc_v,
                     sem_i0, sem_i1, sem_o0, sem_o1):
        wid = lax.axis_index("s") * SC_CORES + lax.axis_index("c")
        base = wid * tpw
        sem_i = (sem_i0, sem_i1)
        sem_o = (sem_o0, sem_o1)

        pltpu.sync_copy(idx_hbm.at[pl.ds(base * 8, tpw * 8)], idx_v)
        pltpu.sync_copy(
            w_hbm.at[pl.ds(base * TOP_K * 16, tpw * TOP_K * 16)], w_v)

        def cp_in(t, buf):
            return pltpu.make_async_copy(
                v_hbm.at[idx_v.at[pl.ds(t * 8, TOP_K)]], rows_v.at[buf],
                sem_i[buf])

        def cp_out(t, buf):
            return pltpu.make_async_copy(
                acc_v.at[buf], out_hbm.at[base + t], sem_o[buf])

        cp_in(0, 0).start()

        @pl.loop(0, tpw, step=2)
        def _token_pair(t):
            for b in (0, 1):
                tok = t + b

                @pl.when(tok + 1 < tpw)
                def _():
                    cp_in(tok + 1, 1 - b).start()

                cp_in(tok, b).wait()

                @pl.when(tok >= 2)
                def _():
                    cp_out(tok - 2, b).wait()

                wv = [w_v[pl.ds((tok * TOP_K + k) * 16, 16)]
                      for k in range(TOP_K)]

                @pl.loop(0, PROW // 16, unroll=16)
                def _chunk(j):
                    sl = pl.ds(j * 16, 16)
                    sa = jnp.zeros((16,), jnp.int32)
                    sb = jnp.zeros((16,), jnp.int32)
                    for k in range(TOP_K):
                        r = rows_v[b, k, sl]
                        sa = sa + (r & 0xFFFF) * wv[k]
                        sb = sb + (r >> 16) * wv[k]
                    sa = (sa - (1 << (15 + W_BITS))) >> W_BITS
                    sb = sb >> W_BITS
                    acc_v[b, sl] = (sb << 16) | (sa & 0xFFFF)

                cp_out(tok, b).start()

        cp_out(tpw - 2, 0).wait()
        cp_out(tpw - 1, 1).wait()

    return pl.kernel(
        _gather_body,
        out_type=jax.ShapeDtypeStruct((nt, PROW), jnp.int32),
        mesh=plsc.VectorSubcoreMesh(core_axis_name="c", subcore_axis_name="s",
                                    num_cores=SC_CORES,
                                    num_subcores=SC_SUBCORES),
        scratch_types=[
            pltpu.VMEM((tpw * 8,), jnp.int32),
            pltpu.VMEM((tpw * TOP_K * 16,), jnp.int32),
            pltpu.VMEM((2, TOP_K, PROW), jnp.int32),
            pltpu.VMEM((2, PROW), jnp.int32),
            pltpu.SemaphoreType.DMA,
            pltpu.SemaphoreType.DMA,
            pltpu.SemaphoreType.DMA,
            pltpu.SemaphoreType.DMA,
        ],
    )(v, idx, w)


TB_C = 256


def _proj_body(final, u_ref, p_ref, hre_ref, him_ref, hbl_ref, *rest):
    if final:
        ph0_ref, x0_ref, scal_ref, out_ref, ph_ref = rest
    else:
        out_ref, ph_ref = rest
    pr = p_ref[:, :DIM]
    pi = p_ref[:, DIM:]
    ar = jnp.zeros((TB_C, DIM), jnp.float32)
    ai = jnp.zeros((TB_C, DIM), jnp.float32)
    inv_s2 = 2.0 ** (-2 * V_BITS)
    for r in range(RANK):
        rp = u_ref[:, r * PR:(r + 1) * PR]
        ur = ((rp << 16) >> 16).astype(jnp.float32)
        ui = (rp >> 16).astype(jnp.float32)
        nrm2 = jnp.sum(ur * ur + ui * ui, axis=1) * inv_s2
        q = inv_s2 / jnp.maximum(nrm2, 1e-6)
        cr = jnp.sum(ur * pr + ui * pi, axis=1) * q
        ci = jnp.sum(ur * pi - ui * pr, axis=1) * q
        ar = ar + cr[:, None] * ur - ci[:, None] * ui
        ai = ai + ci[:, None] * ur + cr[:, None] * ui
    sq = jnp.sum(ar * ar + ai * ai, axis=1)
    scale = lax.rsqrt(jnp.maximum(sq, 1e-6))
    psr = ar * scale[:, None]
    psi_ = ai * scale[:, None]
    ls = [jnp.sum(psr * hre_ref[j][None, :] + psi_ * him_ref[j][None, :],
                  axis=1) + hbl_ref[j]
          for j in range(3)]
    m = jnp.maximum(jnp.maximum(ls[0], ls[1]), ls[2])
    e0 = jnp.exp(ls[0] - m)
    ph = e0 / (e0 + jnp.exp(ls[1] - m) + jnp.exp(ls[2] - m))
    ph_ref[...] = ph
    psin = jnp.concatenate([psr, psi_], axis=1)
    if final:
        ph0 = ph0_ref[...]
        still = (ph0 < THRESH).astype(jnp.float32)
        w_a = jnp.where(ph0 >= THRESH, 1.0, ph0)
        w_b = (1.0 - ph0) * still
        acc = (w_a[:, None] * p_ref[...] + w_b[:, None] * psin) * scal_ref[1]
        x0 = x0_ref[...]
        out_ref[...] = x0 + scal_ref[0] * (acc - x0)
    else:
        out_ref[...] = psin


def _project(u, p, hre, him, hbl, ph0=None, x0=None, scal=None, nt=BT):
    final = ph0 is not None
    grid = nt // TB_C
    tok2 = pl.BlockSpec((TB_C, FD), lambda i: (i, 0))
    in_specs = [
        pl.BlockSpec((TB_C, PROW), lambda i: (i, 0)),
        tok2,
        pl.BlockSpec((8, DIM), lambda i: (0, 0)),
        pl.BlockSpec((8, DIM), lambda i: (0, 0)),
        pl.BlockSpec(memory_space=pltpu.SMEM),
    ]
    args = [u, p, hre, him, hbl]
    if final:
        in_specs += [pl.BlockSpec((TB_C,), lambda i: (i,)), tok2,
                     pl.BlockSpec(memory_space=pltpu.SMEM)]
        args += [ph0, x0, scal]
    return pl.pallas_call(
        functools.partial(_proj_body, final),
        grid=(grid,),
        in_specs=in_specs,
        out_specs=[tok2, pl.BlockSpec((TB_C,), lambda i: (i,))],
        out_shape=[
            jax.ShapeDtypeStruct((nt, FD), jnp.float32),
            jax.ShapeDtypeStruct((nt,), jnp.float32),
        ],
        compiler_params=pltpu.CompilerParams(
            dimension_semantics=("arbitrary",)),
    )(*args)


def _route(idx8, w8, nt):
    return idx8.reshape(nt * 8), w8.reshape(nt * TOP_K * 16)


def kernel(psi, bank_keys, bank_values, halt_w_logits, halt_b_logits,
           halt_w_abg, halt_b_abg, head_mix, out_scale):
    psi3 = psi.reshape(BT, DIM, 2)
    x0 = jnp.concatenate([psi3[..., 0], psi3[..., 1]], axis=1)
    k2 = jnp.concatenate([bank_keys[..., 0], bank_keys[..., 1]], axis=1)
    kt = k2.T
    vi = jnp.clip(jnp.round(bank_values.reshape(BANK, PROW, 2)
                            * (2.0 ** V_BITS)), -32767, 32767).astype(jnp.int32)
    v = (vi[..., 1] << 16) | ((vi[..., 0] + 32768) & 0xFFFF)
    hwl3 = halt_w_logits.reshape(DIM, 2, 3)
    hre = jnp.zeros((8, DIM), jnp.float32).at[:3].set(hwl3[:, 0, :].T)
    him = jnp.zeros((8, DIM), jnp.float32).at[:3].set(hwl3[:, 1, :].T)
    head_w = jax.nn.softmax(head_mix)[0]
    scal = jnp.stack([out_scale.astype(jnp.float32), head_w])

    NCH = 2
    HN = BT // NCH
    hbl = halt_b_logits

    outs, ph0s, ph1s = [], [], []
    halves = []
    for h in range(NCH):
        x0_h = x0[h * HN:(h + 1) * HN]
        idx8, w8 = _scores_topk(x0_h, kt, nt=HN)
        idxf, wef = _route(idx8, w8, HN)
        u1 = _gather(v, idxf, wef, nt=HN)
        psi1_h, ph0_h = _project(u1, x0_h, hre, him, hbl, nt=HN)
        halves.append((x0_h, psi1_h, ph0_h))
    for h in range(NCH):
        x0_h, psi1_h, ph0_h = halves[h]
        idx8b, w8b = _scores_topk(psi1_h, kt, nt=HN)
        idxb, web = _route(idx8b, w8b, HN)
        u2 = _gather(v, idxb, web, nt=HN)
        out_h, ph1_h = _project(u2, psi1_h, hre, him, hbl,
                                ph0=ph0_h, x0=x0_h, scal=scal, nt=HN)
        outs.append(out_h)
        ph0s.append(ph0_h)
        ph1s.append(ph1_h)

    ph0 = jnp.concatenate(ph0s)
    ph1 = jnp.concatenate(ph1s)
    psi_out = jnp.concatenate(outs, axis=0)
    still = (ph0 < THRESH).astype(jnp.float32)
    cost = jnp.mean(ph0 + ph1 * still)
    out3 = jnp.stack([psi_out[:, :DIM], psi_out[:, DIM:]], axis=-1)
    return out3.reshape(B, T, DIM, 2), cost
